# baseline (speedup 1.0000x reference)
import functools

import jax
import jax.numpy as jnp
from jax import lax
from jax.experimental import pallas as pl
from jax.experimental.pallas import tpu as pltpu
from jax.experimental.pallas import tpu_sc as plsc

N = 10000
E = 320000
IN = 128
H = 256
OUT = 128

NCORES = 2
NSUB = 16
NW = NCORES * NSUB
CK = 64
NCH = 160
CH_TOT = NW * NCH
E_PAD = CH_TOT * CK
NP = 10240
RPT = NP // NSUB
NBUF = 4
CA = 264
CB = 56


def _sc_aggregate(with_counts):
    mesh = plsc.VectorSubcoreMesh(
        core_axis_name="c", subcore_axis_name="s",
        num_cores=NCORES, num_subcores=NSUB)

    out_type = [jax.ShapeDtypeStruct((NCORES, NP, 128), jnp.float32)]
    scratch = [
        pltpu.VMEM((NBUF, 2, CK), jnp.int32),
        pltpu.VMEM((NBUF, CK, 128), jnp.float32),
        pltpu.VMEM_SHARED((NP, 128), jnp.float32),
        pltpu.SemaphoreType.DMA((NBUF,)),
        pltpu.SemaphoreType.DMA((NBUF,)),
        pltpu.SemaphoreType.DMA((NBUF,)),
    ]
    if with_counts:
        out_type.append(jax.ShapeDtypeStruct((NW, NP), jnp.float32))
        scratch.append(pltpu.VMEM((NP,), jnp.float32))

    @functools.partial(
        pl.kernel, out_type=out_type, mesh=mesh, scratch_types=scratch,
        compiler_params=pltpu.CompilerParams(needs_layout_passes=False))
    def kfn(table, idxp, out, *rest):
        if with_counts:
            cnt_out, iring, bufs, acc, isem, gsem, ssem, cntv = rest
        else:
            iring, bufs, acc, isem, gsem, ssem = rest
        cid = lax.axis_index("c")
        sid = lax.axis_index("s")
        wid = sid * NCORES + cid

        zv = jnp.zeros((16,), jnp.float32)
        buf0 = bufs.at[0]

        def zb(r, carry):
            for t in range(8):
                buf0[r, pl.ds(t * 16, 16)] = zv
            return carry

        lax.fori_loop(0, CK, zb, 0)
        for t in range(RPT // CK):
            pltpu.sync_copy(buf0, acc.at[pl.ds(sid * RPT + t * CK, CK), :])

        if with_counts:
            def zc(r, carry):
                cntv[pl.ds(r * 16, 16)] = zv
                return carry
            lax.fori_loop(0, NP // 16, zc, 0)

        plsc.subcore_barrier()

        ones16 = jnp.ones((16,), jnp.float32)
        pch = jnp.where(cid == 0, CA, CB)
        base = jnp.where(cid == 0, sid * CA, NSUB * CA + sid * CB)

        def start_i(j, slot):
            pltpu.async_copy(idxp.at[base + j], iring.at[slot],
                             isem.at[slot])

        def start_g(slot):
            pltpu.async_copy(table.at[iring.at[slot, 0]], bufs.at[slot],
                             gsem.at[slot])

        def start_s(slot):
            pltpu.async_copy(bufs.at[slot], acc.at[iring.at[slot, 1]],
                             ssem.at[slot], add=True)

        def wait_i(slot):
            pltpu.make_async_copy(idxp.at[base], iring.at[slot],
                                  isem.at[slot]).wait()

        def wait_g(slot):
            pltpu.make_async_copy(table.at[iring.at[slot, 0]],
                                  bufs.at[slot], gsem.at[slot]).wait()

        def wait_s(slot):
            pltpu.make_async_copy(bufs.at[slot], acc.at[iring.at[slot, 1]],
                                  ssem.at[slot]).wait()

        start_i(0, 0)
        start_i(1, 1)
        wait_i(0)
        start_g(0)

        def body(g, carry):
            for b in range(NBUF):
                j = NBUF * g + b
                s2 = (b + 2) % NBUF
                s1 = (b + 1) % NBUF

                @pl.when(j >= 2)
                def _():
                    wait_s(s2)

                @pl.when(j + 2 < pch)
                def _():
                    start_i(j + 2, s2)

                @pl.when(j + 1 < pch)
                def _():
                    wait_i(s1)
                    start_g(s1)

                wait_g(b)
                start_s(b)

                if with_counts:
                    for t in range(CK // 16):
                        idxv = iring[b, 1, pl.ds(t * 16, 16)]
                        plsc.addupdate_scatter(cntv, [idxv], ones16)
            return carry

        lax.fori_loop(0, pch // NBUF, body, 0)
        wait_s(2)
        wait_s(3)
        plsc.subcore_barrier()

        pltpu.sync_copy(acc.at[pl.ds(sid * RPT, RPT), :],
                        out.at[cid, pl.ds(sid * RPT, RPT), :])
        if with_counts:
            pltpu.sync_copy(cntv, cnt_out.at[wid])

    return kfn


_DN = (((1,), (1,)), ((), ()))
_DT = (((0,), (0,)), ((), ()))


def _tc_dense(pa, cp, x, w1l, b1l, w1r, w2l, b2l, w2r):
    R = 1024

    def body(pa_ref, cp_ref, x_ref, w1l_ref, b1l_ref, w1r_ref, w2l_ref,
             b2l_ref, w2r_ref, p_ref, r_ref, ic_ref):
        cnt = lax.dot_general(cp_ref[...], jnp.ones((NW, 1), jnp.float32),
                              _DT, preferred_element_type=jnp.float32)
        invc = 1.0 / jnp.maximum(cnt, 1.0)
        mean = (pa_ref[0] + pa_ref[1]) * invc
        h = lax.dot_general(mean, w1l_ref[...], _DN,
                            preferred_element_type=jnp.float32)
        h = h + b1l_ref[...]
        h = h + lax.dot_general(x_ref[...], w1r_ref[...], _DN,
                                preferred_element_type=jnp.float32)
        h = jnp.maximum(h, 0.0)
        p_ref[...] = lax.dot_general(h, w2l_ref[...], _DN,
                                     preferred_element_type=jnp.float32)
        r_ref[...] = lax.dot_general(h, w2r_ref[...], _DN,
                                     preferred_element_type=jnp.float32) \
            + b2l_ref[...]
        ic_ref[...] = invc

    return pl.pallas_call(
        body,
        grid=(NP // R,),
        in_specs=[
            pl.BlockSpec((NCORES, R, IN), lambda i: (0, i, 0)),
            pl.BlockSpec((NW, R), lambda i: (0, i)),
            pl.BlockSpec((R, IN), lambda i: (i, 0)),
            pl.BlockSpec((H, IN), lambda i: (0, 0)),
            pl.BlockSpec((1, H), lambda i: (0, 0)),
            pl.BlockSpec((H, IN), lambda i: (0, 0)),
            pl.BlockSpec((OUT, H), lambda i: (0, 0)),
            pl.BlockSpec((1, OUT), lambda i: (0, 0)),
            pl.BlockSpec((OUT, H), lambda i: (0, 0)),
        ],
        out_specs=[
            pl.BlockSpec((R, OUT), lambda i: (i, 0)),
            pl.BlockSpec((R, OUT), lambda i: (i, 0)),
            pl.BlockSpec((R, 1), lambda i: (i, 0)),
        ],
        out_shape=[
            jax.ShapeDtypeStruct((N, OUT), jnp.float32),
            jax.ShapeDtypeStruct((N, OUT), jnp.float32),
            jax.ShapeDtypeStruct((N, 1), jnp.float32),
        ],
    )(pa, cp, x, w1l, b1l, w1r, w2l, b2l, w2r)


def _tc_finish(pb, invc, r):
    R = 1024

    def body(pb_ref, ic_ref, r_ref, o_ref):
        o_ref[...] = (pb_ref[0] + pb_ref[1]) * ic_ref[...] + r_ref[...]

    return pl.pallas_call(
        body,
        grid=(NP // R,),
        in_specs=[
            pl.BlockSpec((NCORES, R, OUT), lambda i: (0, i, 0)),
            pl.BlockSpec((R, 1), lambda i: (i, 0)),
            pl.BlockSpec((R, OUT), lambda i: (i, 0)),
        ],
        out_specs=pl.BlockSpec((R, OUT), lambda i: (i, 0)),
        out_shape=jax.ShapeDtypeStruct((N, OUT), jnp.float32),
    )(pb, invc, r)


@jax.jit
def kernel(x, edge_index, W1l, b1l, W1r, W2l, b2l, W2r):
    src = edge_index[0]
    dst = edge_index[1]
    pad = E_PAD - E
    srcp = jnp.concatenate(
        [src, jnp.zeros((pad,), jnp.int32)]).reshape(CH_TOT, CK)
    dstp = jnp.concatenate(
        [dst, N + jnp.arange(pad, dtype=jnp.int32) % (NP - N)]
    ).reshape(CH_TOT, CK)
    idxp = jnp.stack([srcp, dstp], axis=1)

    pa, cp = _sc_aggregate(True)(x, idxp)
    p, r, invc = _tc_dense(pa, cp, x, W1l, b1l.reshape(1, H), W1r,
                           W2l, b2l.reshape(1, OUT), W2r)
    pb, = _sc_aggregate(False)(p, idxp)
    return _tc_finish(pb, invc, r)


# speedup vs baseline: 1.1093x; 1.1093x over previous
import functools

import jax
import jax.numpy as jnp
from jax import lax
from jax.experimental import pallas as pl
from jax.experimental.pallas import tpu as pltpu
from jax.experimental.pallas import tpu_sc as plsc

N = 10000
E = 320000
IN = 128
H = 256
OUT = 128

NCORES = 2
NSUB = 16
NW = NCORES * NSUB
CK = 64
NCH = 160
CH_TOT = NW * NCH
E_PAD = CH_TOT * CK
NP = 10240
RPT = NP // NSUB
NBUF = 4
CA = 304
CB = 16


def _sc_aggregate(with_counts):
    mesh = plsc.VectorSubcoreMesh(
        core_axis_name="c", subcore_axis_name="s",
        num_cores=NCORES, num_subcores=NSUB)

    out_type = [jax.ShapeDtypeStruct((NCORES, NP, 128), jnp.float32)]
    scratch = [
        pltpu.VMEM((NBUF, 2, CK), jnp.int32),
        pltpu.VMEM((NBUF, CK, 128), jnp.float32),
        pltpu.VMEM_SHARED((NP, 128), jnp.float32),
        pltpu.SemaphoreType.DMA((NBUF,)),
        pltpu.SemaphoreType.DMA((NBUF,)),
        pltpu.SemaphoreType.DMA((NBUF,)),
    ]
    if with_counts:
        out_type.append(jax.ShapeDtypeStruct((NW, NP), jnp.float32))
        scratch.append(pltpu.VMEM((NP,), jnp.float32))

    @functools.partial(
        pl.kernel, out_type=out_type, mesh=mesh, scratch_types=scratch,
        compiler_params=pltpu.CompilerParams(needs_layout_passes=False))
    def kfn(table, idxp, out, *rest):
        if with_counts:
            cnt_out, iring, bufs, acc, isem, gsem, ssem, cntv = rest
        else:
            iring, bufs, acc, isem, gsem, ssem = rest
        cid = lax.axis_index("c")
        sid = lax.axis_index("s")
        wid = sid * NCORES + cid

        zv = jnp.zeros((16,), jnp.float32)
        buf0 = bufs.at[0]

        def zb(r, carry):
            for t in range(8):
                buf0[r, pl.ds(t * 16, 16)] = zv
            return carry

        lax.fori_loop(0, CK, zb, 0)
        for t in range(RPT // CK):
            pltpu.sync_copy(buf0, acc.at[pl.ds(sid * RPT + t * CK, CK), :])

        if with_counts:
            def zc(r, carry):
                cntv[pl.ds(r * 16, 16)] = zv
                return carry
            lax.fori_loop(0, NP // 16, zc, 0)

        plsc.subcore_barrier()

        ones16 = jnp.ones((16,), jnp.float32)
        pch = jnp.where(cid == 0, CA, CB)
        base = jnp.where(cid == 0, sid * CA, NSUB * CA + sid * CB)

        def start_i(j, slot):
            pltpu.async_copy(idxp.at[base + j], iring.at[slot],
                             isem.at[slot])

        def start_g(slot):
            pltpu.async_copy(table.at[iring.at[slot, 0]], bufs.at[slot],
                             gsem.at[slot])

        def start_s(slot):
            pltpu.async_copy(bufs.at[slot], acc.at[iring.at[slot, 1]],
                             ssem.at[slot], add=True)

        def wait_i(slot):
            pltpu.make_async_copy(idxp.at[base], iring.at[slot],
                                  isem.at[slot]).wait()

        def wait_g(slot):
            pltpu.make_async_copy(table.at[iring.at[slot, 0]],
                                  bufs.at[slot], gsem.at[slot]).wait()

        def wait_s(slot):
            pltpu.make_async_copy(bufs.at[slot], acc.at[iring.at[slot, 1]],
                                  ssem.at[slot]).wait()

        start_i(0, 0)
        start_i(1, 1)
        wait_i(0)
        start_g(0)

        def body(g, carry):
            for b in range(NBUF):
                j = NBUF * g + b
                s2 = (b + 2) % NBUF
                s1 = (b + 1) % NBUF

                @pl.when(j >= 2)
                def _():
                    wait_s(s2)

                @pl.when(j + 2 < pch)
                def _():
                    start_i(j + 2, s2)

                @pl.when(j + 1 < pch)
                def _():
                    wait_i(s1)
                    start_g(s1)

                wait_g(b)
                start_s(b)

                if with_counts:
                    for t in range(CK // 16):
                        idxv = iring[b, 1, pl.ds(t * 16, 16)]
                        plsc.addupdate_scatter(cntv, [idxv], ones16)
            return carry

        lax.fori_loop(0, pch // NBUF, body, 0)
        wait_s(2)
        wait_s(3)
        plsc.subcore_barrier()

        pltpu.sync_copy(acc.at[pl.ds(sid * RPT, RPT), :],
                        out.at[cid, pl.ds(sid * RPT, RPT), :])
        if with_counts:
            pltpu.sync_copy(cntv, cnt_out.at[wid])

    return kfn


_DN = (((1,), (1,)), ((), ()))
_DT = (((0,), (0,)), ((), ()))


def _tc_dense(pa, cp, x, w1l, b1l, w1r, w2l, b2l, w2r):
    R = 1024

    def body(pa_ref, cp_ref, x_ref, w1l_ref, b1l_ref, w1r_ref, w2l_ref,
             b2l_ref, w2r_ref, p_ref, r_ref, ic_ref):
        cnt = lax.dot_general(cp_ref[...], jnp.ones((NW, 1), jnp.float32),
                              _DT, preferred_element_type=jnp.float32)
        invc = 1.0 / jnp.maximum(cnt, 1.0)
        mean = (pa_ref[0] + pa_ref[1]) * invc
        h = lax.dot_general(mean, w1l_ref[...], _DN,
                            preferred_element_type=jnp.float32)
        h = h + b1l_ref[...]
        h = h + lax.dot_general(x_ref[...], w1r_ref[...], _DN,
                                preferred_element_type=jnp.float32)
        h = jnp.maximum(h, 0.0)
        p_ref[...] = lax.dot_general(h, w2l_ref[...], _DN,
                                     preferred_element_type=jnp.float32)
        r_ref[...] = lax.dot_general(h, w2r_ref[...], _DN,
                                     preferred_element_type=jnp.float32) \
            + b2l_ref[...]
        ic_ref[...] = invc

    return pl.pallas_call(
        body,
        grid=(NP // R,),
        in_specs=[
            pl.BlockSpec((NCORES, R, IN), lambda i: (0, i, 0)),
            pl.BlockSpec((NW, R), lambda i: (0, i)),
            pl.BlockSpec((R, IN), lambda i: (i, 0)),
            pl.BlockSpec((H, IN), lambda i: (0, 0)),
            pl.BlockSpec((1, H), lambda i: (0, 0)),
            pl.BlockSpec((H, IN), lambda i: (0, 0)),
            pl.BlockSpec((OUT, H), lambda i: (0, 0)),
            pl.BlockSpec((1, OUT), lambda i: (0, 0)),
            pl.BlockSpec((OUT, H), lambda i: (0, 0)),
        ],
        out_specs=[
            pl.BlockSpec((R, OUT), lambda i: (i, 0)),
            pl.BlockSpec((R, OUT), lambda i: (i, 0)),
            pl.BlockSpec((R, 1), lambda i: (i, 0)),
        ],
        out_shape=[
            jax.ShapeDtypeStruct((N, OUT), jnp.float32),
            jax.ShapeDtypeStruct((N, OUT), jnp.float32),
            jax.ShapeDtypeStruct((N, 1), jnp.float32),
        ],
    )(pa, cp, x, w1l, b1l, w1r, w2l, b2l, w2r)


def _tc_finish(pb, invc, r):
    R = 1024

    def body(pb_ref, ic_ref, r_ref, o_ref):
        o_ref[...] = (pb_ref[0] + pb_ref[1]) * ic_ref[...] + r_ref[...]

    return pl.pallas_call(
        body,
        grid=(NP // R,),
        in_specs=[
            pl.BlockSpec((NCORES, R, OUT), lambda i: (0, i, 0)),
            pl.BlockSpec((R, 1), lambda i: (i, 0)),
            pl.BlockSpec((R, OUT), lambda i: (i, 0)),
        ],
        out_specs=pl.BlockSpec((R, OUT), lambda i: (i, 0)),
        out_shape=jax.ShapeDtypeStruct((N, OUT), jnp.float32),
    )(pb, invc, r)


@jax.jit
def kernel(x, edge_index, W1l, b1l, W1r, W2l, b2l, W2r):
    src = edge_index[0]
    dst = edge_index[1]
    pad = E_PAD - E
    srcp = jnp.concatenate(
        [src, jnp.zeros((pad,), jnp.int32)]).reshape(CH_TOT, CK)
    dstp = jnp.concatenate(
        [dst, N + jnp.arange(pad, dtype=jnp.int32) % (NP - N)]
    ).reshape(CH_TOT, CK)
    idxp = jnp.stack([srcp, dstp], axis=1)

    pa, cp = _sc_aggregate(True)(x, idxp)
    p, r, invc = _tc_dense(pa, cp, x, W1l, b1l.reshape(1, H), W1r,
                           W2l, b2l.reshape(1, OUT), W2r)
    pb, = _sc_aggregate(False)(p, idxp)
    return _tc_finish(pb, invc, r)
